# baseline (speedup 1.0000x reference)
import functools

import jax
import jax.numpy as jnp
from jax import lax
from jax.experimental import pallas as pl
from jax.experimental.pallas import tpu as pltpu
from jax.experimental.pallas import tpu_sc as plsc


def _sc_gather(idx, table):
    B, = idx.shape
    V, D = table.shape
    info = plsc.get_sparse_core_info()
    NC, NS = info.num_cores, info.num_subcores
    NW = NC * NS
    b_per_w = B // NW
    mesh = plsc.VectorSubcoreMesh(core_axis_name="c", subcore_axis_name="s")

    @functools.partial(
        pl.kernel,
        mesh=mesh,
        out_type=jax.ShapeDtypeStruct((B, D), table.dtype),
        scratch_types=[
            pltpu.VMEM((b_per_w,), jnp.int32),
            pltpu.VMEM((b_per_w, D), table.dtype),
            pltpu.SemaphoreType.DMA,
        ],
    )
    def k(idx_hbm, table_hbm, out_hbm, idx_v, rows_v, sem):
        wid = lax.axis_index("s") * NC + lax.axis_index("c")
        base = wid * b_per_w
        pltpu.sync_copy(idx_hbm.at[pl.ds(base, b_per_w)], idx_v)
        pltpu.async_copy(table_hbm.at[idx_v], rows_v, sem).wait()
        pltpu.sync_copy(rows_v, out_hbm.at[pl.ds(base, b_per_w)])

    return k(idx, table)


def _tc_project_t(z, out_w, out_b, tile):
    B, D = z.shape
    V, _ = out_w.shape

    def body(z_ref, w_ref, b_ref, o_ref):
        b_col = jnp.transpose(b_ref[...], (1, 0))
        o_ref[...] = lax.dot_general(
            w_ref[...], z_ref[...],
            dimension_numbers=(((1,), (1,)), ((), ())),
            preferred_element_type=jnp.float32,
        ) + b_col

    out_t = pl.pallas_call(
        body,
        grid=(pl.cdiv(V, tile),),
        in_specs=[
            pl.BlockSpec((B, D), lambda i: (0, 0)),
            pl.BlockSpec((tile, D), lambda i: (i, 0)),
            pl.BlockSpec((1, tile), lambda i: (0, i)),
        ],
        out_specs=pl.BlockSpec((tile, B), lambda i: (i, 0)),
        out_shape=jax.ShapeDtypeStruct((V, B), jnp.float32),
    )(z, out_w, out_b.reshape(1, V))
    return out_t.T


def kernel(x, emb_table, out_w, out_b):
    z = _sc_gather(x.astype(jnp.int32), emb_table)
    return _tc_project_t(z, out_w, out_b, tile=5632)


# speedup vs baseline: 1.0036x; 1.0036x over previous
import functools

import jax
import jax.numpy as jnp
from jax import lax
from jax.experimental import pallas as pl
from jax.experimental.pallas import tpu as pltpu
from jax.experimental.pallas import tpu_sc as plsc


def _sc_gather(idx, table):
    B, = idx.shape
    V, D = table.shape
    info = plsc.get_sparse_core_info()
    NC, NS = info.num_cores, info.num_subcores
    NW = NC * NS
    b_per_w = B // NW
    mesh = plsc.VectorSubcoreMesh(core_axis_name="c", subcore_axis_name="s")

    @functools.partial(
        pl.kernel,
        mesh=mesh,
        out_type=jax.ShapeDtypeStruct((B, D), table.dtype),
        scratch_types=[
            pltpu.VMEM((b_per_w,), jnp.int32),
            pltpu.VMEM((b_per_w, D), table.dtype),
            pltpu.SemaphoreType.DMA,
        ],
    )
    def k(idx_hbm, table_hbm, out_hbm, idx_v, rows_v, sem):
        wid = lax.axis_index("s") * NC + lax.axis_index("c")
        base = wid * b_per_w
        pltpu.sync_copy(idx_hbm.at[pl.ds(base, b_per_w)], idx_v)
        pltpu.async_copy(table_hbm.at[idx_v], rows_v, sem).wait()
        pltpu.sync_copy(rows_v, out_hbm.at[pl.ds(base, b_per_w)])

    return k(idx, table)


def _tc_project_t(z, out_w, out_b, tile):
    B, D = z.shape
    V, _ = out_w.shape

    def body(z_ref, w_ref, b_ref, o_ref):
        b_col = jnp.transpose(jnp.reshape(b_ref[...], (1, tile)), (1, 0))
        o_ref[...] = lax.dot_general(
            w_ref[...], z_ref[...],
            dimension_numbers=(((1,), (1,)), ((), ())),
            preferred_element_type=jnp.float32,
        ) + b_col

    out_t = pl.pallas_call(
        body,
        grid=(pl.cdiv(V, tile),),
        in_specs=[
            pl.BlockSpec((B, D), lambda i: (0, 0)),
            pl.BlockSpec((tile, D), lambda i: (i, 0)),
            pl.BlockSpec((tile,), lambda i: (i,)),
        ],
        out_specs=pl.BlockSpec((tile, B), lambda i: (i, 0)),
        out_shape=jax.ShapeDtypeStruct((V, B), jnp.float32),
    )(z, out_w, out_b)
    return out_t.T


def kernel(x, emb_table, out_w, out_b):
    z = _sc_gather(x.astype(jnp.int32), emb_table)
    return _tc_project_t(z, out_w, out_b, tile=5120)


# speedup vs baseline: 1.0105x; 1.0069x over previous
import functools

import jax
import jax.numpy as jnp
from jax import lax
from jax.experimental import pallas as pl
from jax.experimental.pallas import tpu as pltpu
from jax.experimental.pallas import tpu_sc as plsc


def _sc_gather(idx, table):
    B, = idx.shape
    V, D = table.shape
    info = plsc.get_sparse_core_info()
    NC, NS = info.num_cores, info.num_subcores
    NW = NC * NS
    b_per_w = B // NW
    mesh = plsc.VectorSubcoreMesh(core_axis_name="c", subcore_axis_name="s")

    @functools.partial(
        pl.kernel,
        mesh=mesh,
        out_type=jax.ShapeDtypeStruct((B, D), table.dtype),
        scratch_types=[
            pltpu.VMEM((b_per_w,), jnp.int32),
            pltpu.VMEM((b_per_w, D), table.dtype),
            pltpu.SemaphoreType.DMA,
        ],
    )
    def k(idx_hbm, table_hbm, out_hbm, idx_v, rows_v, sem):
        wid = lax.axis_index("s") * NC + lax.axis_index("c")
        base = wid * b_per_w
        pltpu.sync_copy(idx_hbm.at[pl.ds(base, b_per_w)], idx_v)
        pltpu.async_copy(table_hbm.at[idx_v], rows_v, sem).wait()
        pltpu.sync_copy(rows_v, out_hbm.at[pl.ds(base, b_per_w)])

    return k(idx, table)


def _tc_project_t(z, out_w, out_b, tile):
    B, D = z.shape
    V, _ = out_w.shape

    def body(z_ref, w_ref, b_ref, o_ref):
        b_col = jnp.transpose(jnp.reshape(b_ref[...], (1, tile)), (1, 0))
        o_ref[...] = lax.dot_general(
            w_ref[...], z_ref[...],
            dimension_numbers=(((1,), (1,)), ((), ())),
            preferred_element_type=jnp.float32,
        ) + b_col

    out_t = pl.pallas_call(
        body,
        grid=(pl.cdiv(V, tile),),
        in_specs=[
            pl.BlockSpec((B, D), lambda i: (0, 0)),
            pl.BlockSpec((tile, D), lambda i: (i, 0)),
            pl.BlockSpec((tile,), lambda i: (i,)),
        ],
        out_specs=pl.BlockSpec((tile, B), lambda i: (i, 0)),
        out_shape=jax.ShapeDtypeStruct((V, B), jnp.float32),
    )(z, out_w, out_b)
    return out_t.T


def _tc_project_t_manual(z, out_w, out_b, tile, nq):
    B, D = z.shape
    V, _ = out_w.shape
    n = pl.cdiv(V, tile)
    last = V - (n - 1) * tile
    rq, lq = tile // nq, last // nq
    assert tile % (8 * nq) == 0 and last % (8 * nq) == 0

    def body(z_ref, w_ref, b_ref, o_hbm, acc_ref, sems):
        i = pl.program_id(0)
        slot = lax.rem(i, 2)

        @pl.when(i >= 2)
        def _wait_prev():
            for q in range(nq):
                pltpu.make_async_copy(
                    acc_ref.at[slot, pl.ds(q * rq, rq), :],
                    o_hbm.at[pl.ds((i - 2) * tile + q * rq, rq), :],
                    sems.at[slot, q]).wait()

        b_col = jnp.transpose(jnp.reshape(b_ref[...], (1, tile)), (1, 0))
        acc_ref[slot] = lax.dot_general(
            w_ref[...], z_ref[...],
            dimension_numbers=(((1,), (1,)), ((), ())),
            preferred_element_type=jnp.float32,
        ) + b_col

        @pl.when(i < n - 1)
        def _issue_full():
            for q in range(nq):
                pltpu.make_async_copy(
                    acc_ref.at[slot, pl.ds(q * rq, rq), :],
                    o_hbm.at[pl.ds(i * tile + q * rq, rq), :],
                    sems.at[slot, q]).start()

        @pl.when(i == n - 1)
        def _issue_last_and_drain():
            for q in range(nq):
                pltpu.make_async_copy(
                    acc_ref.at[slot, pl.ds(q * lq, lq), :],
                    o_hbm.at[pl.ds(i * tile + q * lq, lq), :],
                    sems.at[slot, q]).start()
            for q in range(nq):
                pltpu.make_async_copy(
                    acc_ref.at[1 - slot, pl.ds(q * rq, rq), :],
                    o_hbm.at[pl.ds((i - 1) * tile + q * rq, rq), :],
                    sems.at[1 - slot, q]).wait()
                pltpu.make_async_copy(
                    acc_ref.at[slot, pl.ds(q * lq, lq), :],
                    o_hbm.at[pl.ds(i * tile + q * lq, lq), :],
                    sems.at[slot, q]).wait()

    out_t = pl.pallas_call(
        body,
        grid=(n,),
        in_specs=[
            pl.BlockSpec((B, D), lambda i: (0, 0)),
            pl.BlockSpec((tile, D), lambda i: (i, 0)),
            pl.BlockSpec((tile,), lambda i: (i,)),
        ],
        out_specs=pl.BlockSpec(memory_space=pl.ANY),
        out_shape=jax.ShapeDtypeStruct((V, B), jnp.float32),
        scratch_shapes=[
            pltpu.VMEM((2, tile, B), jnp.float32),
            pltpu.SemaphoreType.DMA((2, nq)),
        ],
    )(z, out_w, out_b)
    return out_t.T


def kernel(x, emb_table, out_w, out_b):
    z = _sc_gather(x.astype(jnp.int32), emb_table)
    return _tc_project_t_manual(z, out_w, out_b, tile=5120, nq=2)
